# baseline (speedup 1.0000x reference)
import jax
import jax.numpy as jnp
from jax.experimental import pallas as pl
from jax.experimental.pallas import tpu as pltpu

BATCH = 128
D_MODEL = 1024
VOCAB = 100000
TILE_V = 2048


def _linear_tile(x_ref, w_ref, b_ref, o_ref):
    acc = jax.lax.dot_general(
        x_ref[...], w_ref[...],
        dimension_numbers=(((1,), (1,)), ((), ())),
        preferred_element_type=jnp.float32,
    )
    o_ref[...] = acc + b_ref[...]


def kernel(x, W, b):
    b2 = b.reshape(1, VOCAB)
    grid = (pl.cdiv(VOCAB, TILE_V),)
    out = pl.pallas_call(
        _linear_tile,
        grid=grid,
        in_specs=[
            pl.BlockSpec((BATCH, D_MODEL), lambda i: (0, 0)),
            pl.BlockSpec((TILE_V, D_MODEL), lambda i: (i, 0)),
            pl.BlockSpec((1, TILE_V), lambda i: (0, i)),
        ],
        out_specs=pl.BlockSpec((BATCH, TILE_V), lambda i: (0, i)),
        out_shape=jax.ShapeDtypeStruct((BATCH, VOCAB), jnp.float32),
        compiler_params=pltpu.CompilerParams(
            dimension_semantics=("arbitrary",),
        ),
    )(x, W, b2)
    return out
